# baseline (speedup 1.0000x reference)
import functools

import jax
import jax.numpy as jnp
from jax import lax
from jax.experimental import pallas as pl
from jax.experimental.pallas import tpu as pltpu
from jax.experimental.pallas import tpu_sc as plsc

N = 10000
NP = 10240
E = 320000
D = 128
ROWS_PER_SUB = NP // 16
EW = E // 32
CHUNK = 80
NCH = EW // CHUNK

_mesh = plsc.VectorSubcoreMesh(core_axis_name="c", subcore_axis_name="s")
_f32 = jnp.float32
_i32 = jnp.int32


NH = 128 * D


@functools.partial(
    pl.kernel,
    out_type=jax.ShapeDtypeStruct((2, 128, D), _i32),
    mesh=_mesh,
    compiler_params=pltpu.CompilerParams(needs_layout_passes=False),
    scratch_types=[
        pltpu.VMEM_SHARED((128, D), _i32),
        pltpu.VMEM((128, D), _i32),
        pltpu.VMEM((EW,), _i32),
        pltpu.VMEM((8, D), _i32),
        pltpu.VMEM((1, D), _i32),
    ],
)
def _deg_kernel(dst_hbm, out_hbm, hist_s, hist_v, dst_v, stage_v, iota_v):
    c = lax.axis_index("c")
    s = lax.axis_index("s")
    row0 = s * 8

    def zpriv(r, _):
        def zl(i, _):
            hist_v[r, pl.ds(i * 16, 16)] = jnp.zeros((16,), _i32)
            return 0

        lax.fori_loop(0, D // 16, zl, 0)
        return 0

    lax.fori_loop(0, 128, zpriv, 0)

    def zst(r, _):
        def zl(i, _):
            stage_v[r, pl.ds(i * 16, 16)] = jnp.zeros((16,), _i32)
            return 0

        lax.fori_loop(0, D // 16, zl, 0)
        return 0

    lax.fori_loop(0, 8, zst, 0)
    for i in range(D // 16):
        iota_v[0, pl.ds(i * 16, 16)] = lax.iota(_i32, 16) + (i * 16)
    pltpu.sync_copy(dst_hbm.at[c, s], dst_v)
    pltpu.sync_copy(stage_v, hist_s.at[pl.ds(row0, 8)])

    ones16 = jnp.ones((16,), _i32)

    def body(k, _):
        base = k * 32
        iv0 = dst_v[pl.ds(base, 16)]
        iv1 = dst_v[pl.ds(base + 16, 16)]
        plsc.addupdate_scatter(hist_v, [iv0 >> 7, iv0 & 127], ones16)
        plsc.addupdate_scatter(hist_v, [iv1 >> 7, iv1 & 127], ones16)
        return 0

    lax.fori_loop(0, EW // 32, body, 0)
    ivt = dst_v[pl.ds(EW - 16, 16)]
    plsc.addupdate_scatter(hist_v, [ivt >> 7, ivt & 127], ones16)
    plsc.subcore_barrier()
    pltpu.sync_copy(hist_v, hist_s.at[iota_v.at[0]], add=True)
    plsc.subcore_barrier()
    pltpu.sync_copy(hist_s.at[pl.ds(row0, 8)], stage_v)
    pltpu.sync_copy(stage_v, out_hbm.at[c, pl.ds(row0, 8)])


@functools.partial(
    pl.kernel,
    out_type=jax.ShapeDtypeStruct((2, NP, D), _f32),
    mesh=_mesh,
    compiler_params=pltpu.CompilerParams(needs_layout_passes=False),
    scratch_types=[
        pltpu.VMEM_SHARED((NP, D), _f32),
        pltpu.VMEM((EW,), _i32),
        pltpu.VMEM((NCH, CHUNK), _i32),
        pltpu.VMEM((CHUNK, D), _f32),
        pltpu.VMEM((CHUNK, D), _f32),
        pltpu.SemaphoreType.DMA,
        pltpu.SemaphoreType.DMA,
    ],
)
def _agg_kernel(g_hbm, src_hbm, dst_hbm, out_hbm, acc, src_v, dst_v, rows0, rows1, sem0, sem1):
    c = lax.axis_index("c")
    s = lax.axis_index("s")
    row0 = s * ROWS_PER_SUB

    pltpu.sync_copy(g_hbm.at[pl.ds(row0, ROWS_PER_SUB)], acc.at[pl.ds(row0, ROWS_PER_SUB)])
    plsc.subcore_barrier()

    pltpu.sync_copy(src_hbm.at[c, s], src_v)
    pltpu.sync_copy(dst_hbm.at[c, s], dst_v)

    pltpu.async_copy(g_hbm.at[src_v.at[pl.ds(0, CHUNK)]], rows0, sem0)

    def body(i, _):
        j = 2 * i
        pltpu.async_copy(g_hbm.at[src_v.at[pl.ds((j + 1) * CHUNK, CHUNK)]], rows1, sem1)
        pltpu.make_async_copy(g_hbm.at[src_v.at[pl.ds(j * CHUNK, CHUNK)]], rows0, sem0).wait()
        pltpu.sync_copy(rows0, acc.at[dst_v.at[j]], add=True)
        pltpu.async_copy(g_hbm.at[src_v.at[pl.ds((j + 2) * CHUNK, CHUNK)]], rows0, sem0)
        pltpu.make_async_copy(g_hbm.at[src_v.at[pl.ds((j + 1) * CHUNK, CHUNK)]], rows1, sem1).wait()
        pltpu.sync_copy(rows1, acc.at[dst_v.at[j + 1]], add=True)
        return 0

    lax.fori_loop(0, (NCH - 1) // 2, body, 0)
    jt = NCH - 1
    pltpu.make_async_copy(g_hbm.at[src_v.at[pl.ds(jt * CHUNK, CHUNK)]], rows0, sem0).wait()
    pltpu.sync_copy(rows0, acc.at[dst_v.at[jt]], add=True)

    plsc.subcore_barrier()
    pltpu.sync_copy(
        acc.at[pl.ds(row0, ROWS_PER_SUB)], out_hbm.at[c, pl.ds(row0, ROWS_PER_SUB)]
    )


@functools.partial(
    pl.kernel,
    out_type=jax.ShapeDtypeStruct((2, 2, 128, D), _f32),
    mesh=_mesh,
    compiler_params=pltpu.CompilerParams(needs_layout_passes=False),
    scratch_types=[
        pltpu.VMEM_SHARED((2, 128, D), _f32),
        pltpu.VMEM((NP,), _f32),
        pltpu.VMEM((NP,), _f32),
        pltpu.VMEM((128, D), _f32),
        pltpu.VMEM((128, D), _f32),
        pltpu.VMEM((EW,), _i32),
        pltpu.VMEM((EW,), _i32),
        pltpu.VMEM((8, D), _f32),
        pltpu.VMEM((1, D), _i32),
    ],
)
def _agg2_kernel(g2a_hbm, g2b_hbm, src_hbm, dst_hbm, out_hbm,
                 acc_s, ta_v, tb_v, a0_v, a1_v, src_v, dst_v, stage_v, iota_v):
    c = lax.axis_index("c")
    s = lax.axis_index("s")
    row0 = s * 8

    pltpu.sync_copy(g2a_hbm, ta_v)
    pltpu.sync_copy(g2b_hbm, tb_v)
    pltpu.sync_copy(src_hbm.at[c, s], src_v)
    pltpu.sync_copy(dst_hbm.at[c, s], dst_v)
    def zpriv(r, _):
        def zl(i, _):
            a0_v[r, pl.ds(i * 16, 16)] = jnp.zeros((16,), _f32)
            a1_v[r, pl.ds(i * 16, 16)] = jnp.zeros((16,), _f32)
            return 0

        lax.fori_loop(0, D // 16, zl, 0)
        return 0

    lax.fori_loop(0, 128, zpriv, 0)

    def zst(r, _):
        def zl(i, _):
            stage_v[r, pl.ds(i * 16, 16)] = jnp.zeros((16,), _f32)
            return 0

        lax.fori_loop(0, D // 16, zl, 0)
        return 0

    lax.fori_loop(0, 8, zst, 0)
    for i in range(D // 16):
        iota_v[0, pl.ds(i * 16, 16)] = lax.iota(_i32, 16) + (i * 16)
    pltpu.sync_copy(stage_v, acc_s.at[0, pl.ds(row0, 8)])
    pltpu.sync_copy(stage_v, acc_s.at[1, pl.ds(row0, 8)])

    def do16(base):
        sl = pl.ds(base, 16)
        si = src_v[sl]
        di = dst_v[sl]
        r, col = di >> 7, di & 127
        plsc.addupdate_scatter(a0_v, [r, col], plsc.load_gather(ta_v, [si]))
        plsc.addupdate_scatter(a1_v, [r, col], plsc.load_gather(tb_v, [si]))

    def body(k, _):
        do16(k * 32)
        do16(k * 32 + 16)
        return 0

    lax.fori_loop(0, EW // 32, body, 0)
    do16(EW - 16)
    plsc.subcore_barrier()
    pltpu.sync_copy(a0_v, acc_s.at[0].at[iota_v.at[0]], add=True)
    pltpu.sync_copy(a1_v, acc_s.at[1].at[iota_v.at[0]], add=True)
    plsc.subcore_barrier()
    pltpu.sync_copy(acc_s.at[0, pl.ds(row0, 8)], stage_v)
    pltpu.sync_copy(stage_v, out_hbm.at[c, 0, pl.ds(row0, 8)])
    pltpu.sync_copy(acc_s.at[1, pl.ds(row0, 8)], stage_v)
    pltpu.sync_copy(stage_v, out_hbm.at[c, 1, pl.ds(row0, 8)])


def _b1_body(x_ref, w1_ref, h_ref):
    h_ref[...] = jnp.dot(x_ref[...], w1_ref[...], preferred_element_type=_f32)


def _b2_body(h_ref, hist_ref, g_ref, ds_ref):
    deg = (hist_ref[0, :, 0] + hist_ref[1, :, 0] + 1).astype(_f32)
    ds = lax.rsqrt(deg)[:, None]
    g_ref[...] = h_ref[...] * ds
    ds_ref[...] = ds


def _d_body(e_ref, g_ref, ds_ref, b1_ref, w2_ref, g2a_ref, g2b_ref):
    ds = ds_ref[...]
    scat = e_ref[0] + e_ref[1] - g_ref[...]
    x1 = jnp.maximum(scat * ds + b1_ref[...], 0.0)
    p = jnp.dot(x1, w2_ref[...], preferred_element_type=_f32)
    g2a_ref[...] = p[:, :1] * ds
    g2b_ref[...] = p[:, 1:2] * ds


def _f_body(e_ref, g2a_ref, g2b_ref, ds_ref, b2_ref, la_ref, lb_ref):
    ds = ds_ref[...]
    o0 = ds * (e_ref[0, 0] + e_ref[1, 0] + g2a_ref[...]) + b2_ref[0]
    o1 = ds * (e_ref[0, 1] + e_ref[1, 1] + g2b_ref[...]) + b2_ref[1]
    m = jnp.maximum(o0, o1)
    z = m + jnp.log(jnp.exp(o0 - m) + jnp.exp(o1 - m))
    la_ref[...] = o0 - z
    lb_ref[...] = o1 - z


def kernel(x, edge_index, W1, b1, W2, b2):
    f32 = _f32
    src = edge_index[0].astype(_i32)
    dst = edge_index[1].astype(_i32)
    srcf = src.reshape(2, 16, EW)
    dstf = dst.reshape(2, 16, EW)
    dstw = dst.reshape(2, 16, NCH, CHUNK)
    w2p = jnp.zeros((D, D), f32).at[:, :2].set(W2)
    b1r = b1.reshape(1, D)

    h = pl.pallas_call(
        _b1_body,
        grid=(5,),
        in_specs=[
            pl.BlockSpec((N // 5, D), lambda i: (i, 0)),
            pl.BlockSpec((D, D), lambda i: (0, 0)),
        ],
        out_specs=pl.BlockSpec((N // 5, D), lambda i: (i, 0)),
        out_shape=jax.ShapeDtypeStruct((N, D), f32),
    )(x, W1)

    hist = _deg_kernel(dstf)
    hist3 = hist.reshape(2, NH)[:, :NP].reshape(2, NP, 1)

    g, dsv = pl.pallas_call(
        _b2_body,
        grid=(8,),
        in_specs=[
            pl.BlockSpec((NP // 8, D), lambda i: (i, 0)),
            pl.BlockSpec((2, NP // 8, 1), lambda i: (0, i, 0)),
        ],
        out_specs=[
            pl.BlockSpec((NP // 8, D), lambda i: (i, 0)),
            pl.BlockSpec((NP // 8, 1), lambda i: (i, 0)),
        ],
        out_shape=[
            jax.ShapeDtypeStruct((NP, D), f32),
            jax.ShapeDtypeStruct((NP, 1), f32),
        ],
    )(h, hist3)

    S2c = _agg_kernel(g, srcf, dstw)

    g2a, g2b = pl.pallas_call(
        _d_body,
        grid=(8,),
        in_specs=[
            pl.BlockSpec((2, NP // 8, D), lambda i: (0, i, 0)),
            pl.BlockSpec((NP // 8, D), lambda i: (i, 0)),
            pl.BlockSpec((NP // 8, 1), lambda i: (i, 0)),
            pl.BlockSpec((1, D), lambda i: (0, 0)),
            pl.BlockSpec((D, D), lambda i: (0, 0)),
        ],
        out_specs=[
            pl.BlockSpec((NP // 8, 1), lambda i: (i, 0)),
            pl.BlockSpec((NP // 8, 1), lambda i: (i, 0)),
        ],
        out_shape=[
            jax.ShapeDtypeStruct((NP, 1), f32),
            jax.ShapeDtypeStruct((NP, 1), f32),
        ],
    )(S2c, g, dsv, b1r, w2p)

    e2 = _agg2_kernel(g2a.reshape(NP), g2b.reshape(NP), srcf, dstf)
    e2 = e2.reshape(2, 2, NH)[:, :, :NP]

    rows2 = NP // D
    lsa, lsb = pl.pallas_call(
        _f_body,
        grid=(1,),
        in_specs=[
            pl.BlockSpec((2, 2, rows2, D), lambda i: (0, 0, 0, 0)),
            pl.BlockSpec((rows2, D), lambda i: (0, 0)),
            pl.BlockSpec((rows2, D), lambda i: (0, 0)),
            pl.BlockSpec((rows2, D), lambda i: (0, 0)),
            pl.BlockSpec(memory_space=pltpu.SMEM),
        ],
        out_specs=[
            pl.BlockSpec((rows2, D), lambda i: (0, 0)),
            pl.BlockSpec((rows2, D), lambda i: (0, 0)),
        ],
        out_shape=[
            jax.ShapeDtypeStruct((rows2, D), f32),
            jax.ShapeDtypeStruct((rows2, D), f32),
        ],
    )(
        e2.reshape(2, 2, rows2, D),
        g2a.reshape(rows2, D),
        g2b.reshape(rows2, D),
        dsv.reshape(rows2, D),
        b2,
    )

    return jnp.stack([lsa.reshape(NP), lsb.reshape(NP)], axis=1)[:N]


# speedup vs baseline: 1.0253x; 1.0253x over previous
import functools

import jax
import jax.numpy as jnp
from jax import lax
from jax.experimental import pallas as pl
from jax.experimental.pallas import tpu as pltpu
from jax.experimental.pallas import tpu_sc as plsc

N = 10000
NP = 10240
E = 320000
D = 128
ROWS_PER_SUB = NP // 16
EW = E // 32
CHUNK = 80
NCH = EW // CHUNK

_mesh = plsc.VectorSubcoreMesh(core_axis_name="c", subcore_axis_name="s")
_f32 = jnp.float32
_i32 = jnp.int32


NH = 128 * D


@functools.partial(
    pl.kernel,
    out_type=jax.ShapeDtypeStruct((2, 128, D), _i32),
    mesh=_mesh,
    compiler_params=pltpu.CompilerParams(needs_layout_passes=False),
    scratch_types=[
        pltpu.VMEM_SHARED((128, D), _i32),
        pltpu.VMEM((128, D), _i32),
        pltpu.VMEM((EW,), _i32),
        pltpu.VMEM((8, D), _i32),
        pltpu.VMEM((1, D), _i32),
    ],
)
def _deg_kernel(dst_hbm, out_hbm, hist_s, hist_v, dst_v, stage_v, iota_v):
    c = lax.axis_index("c")
    s = lax.axis_index("s")
    row0 = s * 8

    def zpriv(r, _):
        def zl(i, _):
            hist_v[r, pl.ds(i * 16, 16)] = jnp.zeros((16,), _i32)
            return 0

        lax.fori_loop(0, D // 16, zl, 0)
        return 0

    lax.fori_loop(0, 128, zpriv, 0)

    def zst(r, _):
        def zl(i, _):
            stage_v[r, pl.ds(i * 16, 16)] = jnp.zeros((16,), _i32)
            return 0

        lax.fori_loop(0, D // 16, zl, 0)
        return 0

    lax.fori_loop(0, 8, zst, 0)
    for i in range(D // 16):
        iota_v[0, pl.ds(i * 16, 16)] = lax.iota(_i32, 16) + (i * 16)
    pltpu.sync_copy(dst_hbm.at[c, s], dst_v)
    pltpu.sync_copy(stage_v, hist_s.at[pl.ds(row0, 8)])

    ones16 = jnp.ones((16,), _i32)

    def body(k, _):
        base = k * 32
        iv0 = dst_v[pl.ds(base, 16)]
        iv1 = dst_v[pl.ds(base + 16, 16)]
        plsc.addupdate_scatter(hist_v, [iv0 >> 7, iv0 & 127], ones16)
        plsc.addupdate_scatter(hist_v, [iv1 >> 7, iv1 & 127], ones16)
        return 0

    lax.fori_loop(0, EW // 32, body, 0)
    ivt = dst_v[pl.ds(EW - 16, 16)]
    plsc.addupdate_scatter(hist_v, [ivt >> 7, ivt & 127], ones16)
    plsc.subcore_barrier()
    pltpu.sync_copy(hist_v, hist_s.at[iota_v.at[0]], add=True)
    plsc.subcore_barrier()
    pltpu.sync_copy(hist_s.at[pl.ds(row0, 8)], stage_v)
    pltpu.sync_copy(stage_v, out_hbm.at[c, pl.ds(row0, 8)])


@functools.partial(
    pl.kernel,
    out_type=jax.ShapeDtypeStruct((2, NP, D), _f32),
    mesh=_mesh,
    compiler_params=pltpu.CompilerParams(needs_layout_passes=False),
    scratch_types=[
        pltpu.VMEM_SHARED((NP, D), _f32),
        pltpu.VMEM((EW,), _i32),
        pltpu.VMEM((NCH, CHUNK), _i32),
        pltpu.VMEM((CHUNK, D), _f32),
        pltpu.VMEM((CHUNK, D), _f32),
        pltpu.SemaphoreType.DMA,
        pltpu.SemaphoreType.DMA,
    ],
)
def _agg_kernel(g_hbm, src_hbm, dst_hbm, out_hbm, acc, src_v, dst_v, rows0, rows1, sem0, sem1):
    c = lax.axis_index("c")
    s = lax.axis_index("s")
    row0 = s * ROWS_PER_SUB

    pltpu.sync_copy(g_hbm.at[pl.ds(row0, ROWS_PER_SUB)], acc.at[pl.ds(row0, ROWS_PER_SUB)])
    plsc.subcore_barrier()

    pltpu.sync_copy(src_hbm.at[c, s], src_v)
    pltpu.sync_copy(dst_hbm.at[c, s], dst_v)

    pltpu.async_copy(g_hbm.at[src_v.at[pl.ds(0, CHUNK)]], rows0, sem0)

    def body(i, _):
        j = 2 * i
        pltpu.async_copy(g_hbm.at[src_v.at[pl.ds((j + 1) * CHUNK, CHUNK)]], rows1, sem1)
        pltpu.make_async_copy(g_hbm.at[src_v.at[pl.ds(j * CHUNK, CHUNK)]], rows0, sem0).wait()
        pltpu.sync_copy(rows0, acc.at[dst_v.at[j]], add=True)
        pltpu.async_copy(g_hbm.at[src_v.at[pl.ds((j + 2) * CHUNK, CHUNK)]], rows0, sem0)
        pltpu.make_async_copy(g_hbm.at[src_v.at[pl.ds((j + 1) * CHUNK, CHUNK)]], rows1, sem1).wait()
        pltpu.sync_copy(rows1, acc.at[dst_v.at[j + 1]], add=True)
        return 0

    lax.fori_loop(0, (NCH - 1) // 2, body, 0)
    jt = NCH - 1
    pltpu.make_async_copy(g_hbm.at[src_v.at[pl.ds(jt * CHUNK, CHUNK)]], rows0, sem0).wait()
    pltpu.sync_copy(rows0, acc.at[dst_v.at[jt]], add=True)

    plsc.subcore_barrier()
    pltpu.sync_copy(
        acc.at[pl.ds(row0, ROWS_PER_SUB)], out_hbm.at[c, pl.ds(row0, ROWS_PER_SUB)]
    )


@functools.partial(
    pl.kernel,
    out_type=jax.ShapeDtypeStruct((2, 2, 128, D), _f32),
    mesh=_mesh,
    compiler_params=pltpu.CompilerParams(needs_layout_passes=False),
    scratch_types=[
        pltpu.VMEM_SHARED((2, 128, D), _f32),
        pltpu.VMEM((NP,), _f32),
        pltpu.VMEM((NP,), _f32),
        pltpu.VMEM((128, D), _f32),
        pltpu.VMEM((128, D), _f32),
        pltpu.VMEM((EW,), _i32),
        pltpu.VMEM((EW,), _i32),
        pltpu.VMEM((8, D), _f32),
        pltpu.VMEM((1, D), _i32),
    ],
)
def _agg2_kernel(g2a_hbm, g2b_hbm, src_hbm, dst_hbm, out_hbm,
                 acc_s, ta_v, tb_v, a0_v, a1_v, src_v, dst_v, stage_v, iota_v):
    c = lax.axis_index("c")
    s = lax.axis_index("s")
    row0 = s * 8

    pltpu.sync_copy(g2a_hbm, ta_v)
    pltpu.sync_copy(g2b_hbm, tb_v)
    pltpu.sync_copy(src_hbm.at[c, s], src_v)
    pltpu.sync_copy(dst_hbm.at[c, s], dst_v)
    def zpriv(r, _):
        def zl(i, _):
            a0_v[r, pl.ds(i * 16, 16)] = jnp.zeros((16,), _f32)
            a1_v[r, pl.ds(i * 16, 16)] = jnp.zeros((16,), _f32)
            return 0

        lax.fori_loop(0, D // 16, zl, 0)
        return 0

    lax.fori_loop(0, 128, zpriv, 0)

    def zst(r, _):
        def zl(i, _):
            stage_v[r, pl.ds(i * 16, 16)] = jnp.zeros((16,), _f32)
            return 0

        lax.fori_loop(0, D // 16, zl, 0)
        return 0

    lax.fori_loop(0, 8, zst, 0)
    for i in range(D // 16):
        iota_v[0, pl.ds(i * 16, 16)] = lax.iota(_i32, 16) + (i * 16)
    pltpu.sync_copy(stage_v, acc_s.at[0, pl.ds(row0, 8)])
    pltpu.sync_copy(stage_v, acc_s.at[1, pl.ds(row0, 8)])

    def do16(base):
        sl = pl.ds(base, 16)
        si = src_v[sl]
        di = dst_v[sl]
        r, col = di >> 7, di & 127
        plsc.addupdate_scatter(a0_v, [r, col], plsc.load_gather(ta_v, [si]))
        plsc.addupdate_scatter(a1_v, [r, col], plsc.load_gather(tb_v, [si]))

    def body(k, _):
        do16(k * 32)
        do16(k * 32 + 16)
        return 0

    lax.fori_loop(0, EW // 32, body, 0)
    do16(EW - 16)
    plsc.subcore_barrier()
    pltpu.sync_copy(a0_v, acc_s.at[0].at[iota_v.at[0]], add=True)
    pltpu.sync_copy(a1_v, acc_s.at[1].at[iota_v.at[0]], add=True)
    plsc.subcore_barrier()
    pltpu.sync_copy(acc_s.at[0, pl.ds(row0, 8)], stage_v)
    pltpu.sync_copy(stage_v, out_hbm.at[c, 0, pl.ds(row0, 8)])
    pltpu.sync_copy(acc_s.at[1, pl.ds(row0, 8)], stage_v)
    pltpu.sync_copy(stage_v, out_hbm.at[c, 1, pl.ds(row0, 8)])


def _b_body(x_ref, w1_ref, hist_ref, g_ref, ds_ref):
    deg = (hist_ref[0, :, 0] + hist_ref[1, :, 0] + 1).astype(_f32)
    ds = lax.rsqrt(deg)[:, None]
    h = jnp.dot(x_ref[...], w1_ref[...], preferred_element_type=_f32)
    g_ref[pl.ds(0, N)] = h * ds[:N]
    g_ref[pl.ds(N, NP - N)] = jnp.zeros((NP - N, D), _f32)
    ds_ref[...] = ds


def _d_body(e_ref, g_ref, ds_ref, b1_ref, w2_ref, g2a_ref, g2b_ref):
    ds = ds_ref[...]
    scat = e_ref[0] + e_ref[1] - g_ref[...]
    x1 = jnp.maximum(scat * ds + b1_ref[...], 0.0)
    p = jnp.dot(x1, w2_ref[...], preferred_element_type=_f32)
    g2a_ref[...] = p[:, :1] * ds
    g2b_ref[...] = p[:, 1:2] * ds


def _f_body(e_ref, g2a_ref, g2b_ref, ds_ref, b2_ref, la_ref, lb_ref):
    ds = ds_ref[...]
    o0 = ds * (e_ref[0, 0] + e_ref[1, 0] + g2a_ref[...]) + b2_ref[0]
    o1 = ds * (e_ref[0, 1] + e_ref[1, 1] + g2b_ref[...]) + b2_ref[1]
    m = jnp.maximum(o0, o1)
    z = m + jnp.log(jnp.exp(o0 - m) + jnp.exp(o1 - m))
    la_ref[...] = o0 - z
    lb_ref[...] = o1 - z


def kernel(x, edge_index, W1, b1, W2, b2):
    f32 = _f32
    src = edge_index[0].astype(_i32)
    dst = edge_index[1].astype(_i32)
    srcf = src.reshape(2, 16, EW)
    dstf = dst.reshape(2, 16, EW)
    dstw = dst.reshape(2, 16, NCH, CHUNK)
    w2p = jnp.zeros((D, D), f32).at[:, :2].set(W2)
    b1r = b1.reshape(1, D)

    hist = _deg_kernel(dstf)
    hist3 = hist.reshape(2, NH)[:, :NP].reshape(2, NP, 1)

    g, dsv = pl.pallas_call(
        _b_body,
        grid=(1,),
        in_specs=[
            pl.BlockSpec((N, D), lambda i: (0, 0)),
            pl.BlockSpec((D, D), lambda i: (0, 0)),
            pl.BlockSpec((2, NP, 1), lambda i: (0, 0, 0)),
        ],
        out_specs=[
            pl.BlockSpec((NP, D), lambda i: (0, 0)),
            pl.BlockSpec((NP, 1), lambda i: (0, 0)),
        ],
        out_shape=[
            jax.ShapeDtypeStruct((NP, D), f32),
            jax.ShapeDtypeStruct((NP, 1), f32),
        ],
    )(x, W1, hist3)

    S2c = _agg_kernel(g, srcf, dstw)

    g2a, g2b = pl.pallas_call(
        _d_body,
        grid=(1,),
        in_specs=[
            pl.BlockSpec((2, NP, D), lambda i: (0, 0, 0)),
            pl.BlockSpec((NP, D), lambda i: (0, 0)),
            pl.BlockSpec((NP, 1), lambda i: (0, 0)),
            pl.BlockSpec((1, D), lambda i: (0, 0)),
            pl.BlockSpec((D, D), lambda i: (0, 0)),
        ],
        out_specs=[
            pl.BlockSpec((NP, 1), lambda i: (0, 0)),
            pl.BlockSpec((NP, 1), lambda i: (0, 0)),
        ],
        out_shape=[
            jax.ShapeDtypeStruct((NP, 1), f32),
            jax.ShapeDtypeStruct((NP, 1), f32),
        ],
    )(S2c, g, dsv, b1r, w2p)

    e2 = _agg2_kernel(g2a.reshape(NP), g2b.reshape(NP), srcf, dstf)
    e2 = e2.reshape(2, 2, NH)[:, :, :NP]

    rows2 = NP // D
    lsa, lsb = pl.pallas_call(
        _f_body,
        grid=(1,),
        in_specs=[
            pl.BlockSpec((2, 2, rows2, D), lambda i: (0, 0, 0, 0)),
            pl.BlockSpec((rows2, D), lambda i: (0, 0)),
            pl.BlockSpec((rows2, D), lambda i: (0, 0)),
            pl.BlockSpec((rows2, D), lambda i: (0, 0)),
            pl.BlockSpec(memory_space=pltpu.SMEM),
        ],
        out_specs=[
            pl.BlockSpec((rows2, D), lambda i: (0, 0)),
            pl.BlockSpec((rows2, D), lambda i: (0, 0)),
        ],
        out_shape=[
            jax.ShapeDtypeStruct((rows2, D), f32),
            jax.ShapeDtypeStruct((rows2, D), f32),
        ],
    )(
        e2.reshape(2, 2, rows2, D),
        g2a.reshape(rows2, D),
        g2b.reshape(rows2, D),
        dsv.reshape(rows2, D),
        b2,
    )

    return jnp.stack([lsa.reshape(NP), lsb.reshape(NP)], axis=1)[:N]
